# baseline (device time: 21578 ns/iter reference)
import jax
import jax.numpy as jnp
from jax import lax
from jax.experimental import pallas as pl
from jax.experimental.pallas import tpu as pltpu

N_DEV = 4
B_LOC = 2
SQ = 128
SKV = 128
HB = 64
HQ = 16
H_LOC = 4
H_PAIR = 2
DH = 64
D_MODEL = 512
HD_LOC = H_LOC * DH
HD_HALF = H_PAIR * DH

BF16 = jnp.bfloat16
F32 = jnp.float32


def kernel(x, Wq, K_ext, V_ext, Wo):
    my = lax.axis_index("i")

    k_loc = lax.dynamic_slice_in_dim(
        K_ext.reshape(8, SKV, HQ * DH), B_LOC * my, B_LOC, axis=0).astype(BF16)
    v_loc = lax.dynamic_slice_in_dim(
        V_ext.reshape(8, SKV, HQ * DH), B_LOC * my, B_LOC, axis=0).astype(BF16)
    wq_t = jnp.swapaxes(Wq, 0, 1).astype(BF16)
    wo_b = Wo.astype(BF16)

    def body(x_ref, wq_ref, k_ref, v_ref, wo_ref, out_ref,
             wq_all, wo_all,
             wq_ssem, wq_rsem, wo_ssem, wo_rsem):
        my_pos = lax.axis_index("i")
        left = lax.rem(my_pos + N_DEV - 1, N_DEV)
        right = lax.rem(my_pos + 1, N_DEV)
        jm1 = left
        jp1 = right
        jm2 = lax.rem(my_pos + 2, N_DEV)

        barrier_sem = pltpu.get_barrier_semaphore()
        for nbr in (left, right):
            pl.semaphore_signal(
                barrier_sem, inc=1,
                device_id=(nbr,), device_id_type=pl.DeviceIdType.MESH,
            )
        pl.semaphore_wait(barrier_sem, 2)

        def rdma(src, dst, ssem, rsem, slot, dev):
            return pltpu.make_async_remote_copy(
                src_ref=src, dst_ref=dst,
                send_sem=ssem.at[slot], recv_sem=rsem.at[slot],
                device_id=(dev,), device_id_type=pl.DeviceIdType.MESH,
            )

        def half(buf, j, p):
            return buf.at[j, pl.ds(p * HD_HALF, HD_HALF), :]

        q_r0 = rdma(wq_ref, wq_all.at[my_pos], wq_ssem, wq_rsem, 0, right)
        o_l0 = rdma(wo_ref, wo_all.at[my_pos], wo_ssem, wo_rsem, 1, left)
        o_r0 = rdma(wo_ref, wo_all.at[my_pos], wo_ssem, wo_rsem, 0, right)
        q_l0 = rdma(wq_ref, wq_all.at[my_pos], wq_ssem, wq_rsem, 1, left)
        q_r0.start()
        o_l0.start()
        o_r0.start()
        q_l0.start()

        wq_all[pl.ds(my_pos, 1)] = wq_ref[...][None]
        wo_all[pl.ds(my_pos, 1)] = wo_ref[...][None]

        q_recv0 = rdma(wq_ref, wq_all.at[jm1], wq_ssem, wq_rsem, 0, left)
        q_recv1 = rdma(wq_ref, wq_all.at[jp1], wq_ssem, wq_rsem, 1, right)
        o_recv0 = rdma(wo_ref, wo_all.at[jm1], wo_ssem, wo_rsem, 0, left)
        o_recv1 = rdma(wo_ref, wo_all.at[jp1], wo_ssem, wo_rsem, 1, right)
        q_recv2 = [
            rdma(half(wq_all, jm1, p), half(wq_all, jm2, p),
                 wq_ssem, wq_rsem, 2 + p, left)
            for p in range(2)
        ]
        o_recv2 = [
            rdma(half(wo_all, jp1, p), half(wo_all, jm2, p),
                 wo_ssem, wo_rsem, 2 + p, right)
            for p in range(2)
        ]

        xs = [(x_ref[b] * 0.125).astype(BF16) for b in range(B_LOC)]

        def ctx_pair(j, p):
            wq_jp = wq_all[pl.ds(j, 1), pl.ds(p * HD_HALF, HD_HALF), :
                           ].reshape(HD_HALF, D_MODEL)
            res = []
            for b in range(B_LOC):
                q_blk = lax.dot_general(
                    xs[b], wq_jp, (((1,), (1,)), ((), ())),
                    preferred_element_type=F32,
                ).astype(BF16)
                col = j * HD_LOC + p * HD_HALF
                kbj = k_ref[b, :, pl.ds(col, HD_HALF)]
                vbj = v_ref[b, :, pl.ds(col, HD_HALF)]
                ctx_t, ctx_b = [], []
                for r in range(H_PAIR):
                    sl = slice(r * DH, (r + 1) * DH)
                    k = kbj[:, sl]
                    v = vbj[:, sl]
                    qt = q_blk[:HB, sl]
                    qb = q_blk[HB:, sl]
                    st = lax.dot_general(
                        qt, k[:HB], (((1,), (1,)), ((), ())),
                        preferred_element_type=F32)
                    sb = lax.dot_general(
                        qb, k, (((1,), (1,)), ((), ())),
                        preferred_element_type=F32)
                    et = jnp.exp(st)
                    eb = jnp.exp(sb)
                    rt = 1.0 / jnp.sum(et, axis=-1, keepdims=True)
                    rb = 1.0 / jnp.sum(eb, axis=-1, keepdims=True)
                    ct = lax.dot_general(
                        et.astype(BF16), v[:HB], (((1,), (0,)), ((), ())),
                        preferred_element_type=F32)
                    cb = lax.dot_general(
                        eb.astype(BF16), v, (((1,), (0,)), ((), ())),
                        preferred_element_type=F32)
                    ctx_t.append((ct * rt).astype(BF16))
                    ctx_b.append((cb * rb).astype(BF16))
                res.append((jnp.concatenate(ctx_t, axis=1),
                            jnp.concatenate(ctx_b, axis=1)))
            return res

        def proj_pair(j, p, ctxs, accs):
            wo_jp = wo_all[pl.ds(j, 1), pl.ds(p * HD_HALF, HD_HALF), :
                           ].reshape(HD_HALF, D_MODEL)
            out = []
            for b in range(B_LOC):
                (cat_t, cat_b), (at, ab) = ctxs[b], accs[b]
                out.append((
                    at + lax.dot_general(
                        cat_t, wo_jp, (((1,), (0,)), ((), ())),
                        preferred_element_type=F32),
                    ab + lax.dot_general(
                        cat_b, wo_jp, (((1,), (0,)), ((), ())),
                        preferred_element_type=F32),
                ))
            return out

        def block(j, accs):
            for p in range(2):
                accs = proj_pair(j, p, ctx_pair(j, p), accs)
            return accs

        accs = [(jnp.zeros((HB, D_MODEL), F32),
                 jnp.zeros((HB, D_MODEL), F32)) for _ in range(B_LOC)]

        accs = block(my_pos, accs)

        q_recv0.wait_recv()
        q_f = [
            rdma(half(wq_all, jm1, p), half(wq_all, jm1, p),
                 wq_ssem, wq_rsem, 2 + p, right)
            for p in range(2)
        ]
        for d in q_f:
            d.start()
        o_recv1.wait_recv()
        o_f = [
            rdma(half(wo_all, jp1, p), half(wo_all, jp1, p),
                 wo_ssem, wo_rsem, 2 + p, left)
            for p in range(2)
        ]
        for d in o_f:
            d.start()

        c_jm1 = [ctx_pair(jm1, p) for p in range(2)]
        o_recv0.wait_recv()
        for p in range(2):
            accs = proj_pair(jm1, p, c_jm1[p], accs)

        q_recv1.wait_recv()
        accs = block(jp1, accs)

        c_jm2 = []
        for p in range(2):
            q_recv2[p].wait_recv()
            c_jm2.append(ctx_pair(jm2, p))
        for p in range(2):
            o_recv2[p].wait_recv()
            accs = proj_pair(jm2, p, c_jm2[p], accs)

        for b in range(B_LOC):
            out_ref[b] = jnp.concatenate([accs[b][0], accs[b][1]], axis=0)

        for d in [q_r0, q_l0, o_r0, o_l0] + q_f + o_f:
            d.wait_send()

    return pl.pallas_call(
        body,
        out_shape=jax.ShapeDtypeStruct((B_LOC, SQ, D_MODEL), F32),
        in_specs=[
            pl.BlockSpec(memory_space=pltpu.VMEM),
            pl.BlockSpec(memory_space=pltpu.VMEM),
            pl.BlockSpec(memory_space=pltpu.VMEM),
            pl.BlockSpec(memory_space=pltpu.VMEM),
            pl.BlockSpec(memory_space=pltpu.VMEM),
        ],
        out_specs=pl.BlockSpec(memory_space=pltpu.VMEM),
        scratch_shapes=[
            pltpu.VMEM((N_DEV, HD_LOC, D_MODEL), BF16),
            pltpu.VMEM((N_DEV, HD_LOC, D_MODEL), BF16),
            pltpu.SemaphoreType.DMA((4,)),
            pltpu.SemaphoreType.DMA((4,)),
            pltpu.SemaphoreType.DMA((4,)),
            pltpu.SemaphoreType.DMA((4,)),
        ],
        compiler_params=pltpu.CompilerParams(collective_id=0),
    )(x, wq_t, k_loc, v_loc, wo_b)


# device time: 20691 ns/iter; 1.0429x vs baseline; 1.0429x over previous
import jax
import jax.numpy as jnp
from jax import lax
from jax.experimental import pallas as pl
from jax.experimental.pallas import tpu as pltpu

N_DEV = 4
B_LOC = 2
SQ = 128
SKV = 128
HB = 64
HQ = 16
H_LOC = 4
DH = 64
D_MODEL = 512
HD_LOC = H_LOC * DH

BF16 = jnp.bfloat16
F32 = jnp.float32


def kernel(x, Wq, K_ext, V_ext, Wo):
    my = lax.axis_index("i")

    k_loc = lax.dynamic_slice_in_dim(
        K_ext.reshape(8, SKV, HQ * DH), B_LOC * my, B_LOC, axis=0).astype(BF16)
    v_loc = lax.dynamic_slice_in_dim(
        V_ext.reshape(8, SKV, HQ * DH), B_LOC * my, B_LOC, axis=0).astype(BF16)
    wq_b = Wq.astype(BF16)
    wo_b = Wo.astype(BF16)

    def body(x_ref, wq_ref, k_ref, v_ref, wo_ref, out_ref,
             wq_all, wo_all,
             wq_ssem, wq_rsem, wo_ssem, wo_rsem):
        my_pos = lax.axis_index("i")
        left = lax.rem(my_pos + N_DEV - 1, N_DEV)
        right = lax.rem(my_pos + 1, N_DEV)
        jm1 = left
        jp1 = right
        jm2 = lax.rem(my_pos + 2, N_DEV)

        barrier_sem = pltpu.get_barrier_semaphore()
        for nbr in (left, right):
            pl.semaphore_signal(
                barrier_sem, inc=1,
                device_id=(nbr,), device_id_type=pl.DeviceIdType.MESH,
            )
        pl.semaphore_wait(barrier_sem, 2)

        def rdma(src, dst, ssem, rsem, slot, dev):
            return pltpu.make_async_remote_copy(
                src_ref=src, dst_ref=dst,
                send_sem=ssem.at[slot], recv_sem=rsem.at[slot],
                device_id=(dev,), device_id_type=pl.DeviceIdType.MESH,
            )

        def o_half(j, p):
            return wo_all.at[j, pl.ds(p * HD_LOC // 2, HD_LOC // 2), :]

        q_r0 = rdma(wq_ref, wq_all.at[my_pos], wq_ssem, wq_rsem, 0, right)
        o_l0 = rdma(wo_ref, wo_all.at[my_pos], wo_ssem, wo_rsem, 1, left)
        o_r0 = rdma(wo_ref, wo_all.at[my_pos], wo_ssem, wo_rsem, 0, right)
        q_l0 = rdma(wq_ref, wq_all.at[my_pos], wq_ssem, wq_rsem, 1, left)
        q_r0.start()
        o_l0.start()
        o_r0.start()
        q_l0.start()

        wq_all[pl.ds(my_pos, 1)] = wq_ref[...][None]
        wo_all[pl.ds(my_pos, 1)] = wo_ref[...][None]

        q_recv0 = rdma(wq_ref, wq_all.at[jm1], wq_ssem, wq_rsem, 0, left)
        q_recv1 = rdma(wq_ref, wq_all.at[jp1], wq_ssem, wq_rsem, 1, right)
        q_recv2 = rdma(wq_ref, wq_all.at[jm2], wq_ssem, wq_rsem, 2, left)
        o_recv0 = rdma(wo_ref, wo_all.at[jm1], wo_ssem, wo_rsem, 0, left)
        o_recv1 = rdma(wo_ref, wo_all.at[jp1], wo_ssem, wo_rsem, 1, right)
        o_recv2 = [
            rdma(o_half(jp1, p), o_half(jm2, p), wo_ssem, wo_rsem, 2 + p,
                 right)
            for p in range(2)
        ]

        xs = [(x_ref[b] * 0.125).astype(BF16) for b in range(B_LOC)]

        def ctx_phase(j):
            wq_j = wq_all[pl.ds(j, 1)].reshape(D_MODEL, HD_LOC)
            res = []
            for b in range(B_LOC):
                q_blk = lax.dot_general(
                    xs[b], wq_j, (((1,), (0,)), ((), ())),
                    preferred_element_type=F32,
                ).astype(BF16)
                kbj = k_ref[b, :, pl.ds(j * HD_LOC, HD_LOC)]
                vbj = v_ref[b, :, pl.ds(j * HD_LOC, HD_LOC)]
                ctx_t, ctx_b = [], []
                for r in range(H_LOC):
                    sl = slice(r * DH, (r + 1) * DH)
                    k = kbj[:, sl]
                    v = vbj[:, sl]
                    qt = q_blk[:HB, sl]
                    qb = q_blk[HB:, sl]
                    st = lax.dot_general(
                        qt, k[:HB], (((1,), (1,)), ((), ())),
                        preferred_element_type=F32)
                    sb = lax.dot_general(
                        qb, k, (((1,), (1,)), ((), ())),
                        preferred_element_type=F32)
                    et = jnp.exp(st)
                    eb = jnp.exp(sb)
                    rt = 1.0 / jnp.sum(et, axis=-1, keepdims=True)
                    rb = 1.0 / jnp.sum(eb, axis=-1, keepdims=True)
                    ct = lax.dot_general(
                        et.astype(BF16), v[:HB], (((1,), (0,)), ((), ())),
                        preferred_element_type=F32)
                    cb = lax.dot_general(
                        eb.astype(BF16), v, (((1,), (0,)), ((), ())),
                        preferred_element_type=F32)
                    ctx_t.append((ct * rt).astype(BF16))
                    ctx_b.append((cb * rb).astype(BF16))
                res.append((jnp.concatenate(ctx_t, axis=1),
                            jnp.concatenate(ctx_b, axis=1)))
            return res

        def proj_half(j, p, ctxs, accs):
            h = HD_LOC // 2
            wo_jp = wo_all[pl.ds(j, 1), pl.ds(p * h, h), :].reshape(h, D_MODEL)
            out = []
            for b in range(B_LOC):
                (cat_t, cat_b), (at, ab) = ctxs[b], accs[b]
                out.append((
                    at + lax.dot_general(
                        cat_t[:, p * h:(p + 1) * h], wo_jp,
                        (((1,), (0,)), ((), ())),
                        preferred_element_type=F32),
                    ab + lax.dot_general(
                        cat_b[:, p * h:(p + 1) * h], wo_jp,
                        (((1,), (0,)), ((), ())),
                        preferred_element_type=F32),
                ))
            return out

        def proj(j, ctxs, accs):
            for p in range(2):
                accs = proj_half(j, p, ctxs, accs)
            return accs

        accs = [(jnp.zeros((HB, D_MODEL), F32),
                 jnp.zeros((HB, D_MODEL), F32)) for _ in range(B_LOC)]

        accs = proj(my_pos, ctx_phase(my_pos), accs)

        q_recv0.wait_recv()
        q_f = rdma(wq_all.at[jm1], wq_all.at[jm1], wq_ssem, wq_rsem, 2, right)
        q_f.start()
        o_recv1.wait_recv()
        o_f = [
            rdma(o_half(jp1, p), o_half(jp1, p), wo_ssem, wo_rsem, 2 + p,
                 left)
            for p in range(2)
        ]
        for d in o_f:
            d.start()

        c_jm1 = ctx_phase(jm1)
        o_recv0.wait_recv()
        accs = proj(jm1, c_jm1, accs)

        q_recv1.wait_recv()
        accs = proj(jp1, ctx_phase(jp1), accs)

        q_recv2.wait_recv()
        c_jm2 = ctx_phase(jm2)
        for p in range(2):
            o_recv2[p].wait_recv()
            accs = proj_half(jm2, p, c_jm2, accs)

        for b in range(B_LOC):
            out_ref[b] = jnp.concatenate([accs[b][0], accs[b][1]], axis=0)

        for d in [q_r0, q_l0, o_r0, o_l0, q_f] + o_f:
            d.wait_send()

    return pl.pallas_call(
        body,
        out_shape=jax.ShapeDtypeStruct((B_LOC, SQ, D_MODEL), F32),
        in_specs=[
            pl.BlockSpec(memory_space=pltpu.VMEM),
            pl.BlockSpec(memory_space=pltpu.VMEM),
            pl.BlockSpec(memory_space=pltpu.VMEM),
            pl.BlockSpec(memory_space=pltpu.VMEM),
            pl.BlockSpec(memory_space=pltpu.VMEM),
        ],
        out_specs=pl.BlockSpec(memory_space=pltpu.VMEM),
        scratch_shapes=[
            pltpu.VMEM((N_DEV, D_MODEL, HD_LOC), BF16),
            pltpu.VMEM((N_DEV, HD_LOC, D_MODEL), BF16),
            pltpu.SemaphoreType.DMA((3,)),
            pltpu.SemaphoreType.DMA((3,)),
            pltpu.SemaphoreType.DMA((4,)),
            pltpu.SemaphoreType.DMA((4,)),
        ],
        compiler_params=pltpu.CompilerParams(collective_id=0),
    )(x, wq_b, k_loc, v_loc, wo_b)


# device time: 18545 ns/iter; 1.1635x vs baseline; 1.1157x over previous
import jax
import jax.numpy as jnp
from jax import lax
from jax.experimental import pallas as pl
from jax.experimental.pallas import tpu as pltpu

N_DEV = 4
B_LOC = 2
SQ = 128
SKV = 128
HB = 64
HQ = 16
H_LOC = 4
DH = 64
D_MODEL = 512
HD_LOC = H_LOC * DH

BF16 = jnp.bfloat16
F32 = jnp.float32


def kernel(x, Wq, K_ext, V_ext, Wo):
    my = lax.axis_index("i")

    k_loc = lax.dynamic_slice_in_dim(
        K_ext.reshape(8, SKV, HQ * DH), B_LOC * my, B_LOC, axis=0).astype(BF16)
    v_loc = lax.dynamic_slice_in_dim(
        V_ext.reshape(8, SKV, HQ * DH), B_LOC * my, B_LOC, axis=0).astype(BF16)

    def body(x_ref, wq_ref, k_ref, v_ref, wo_ref, out_ref,
             wq_all, wo_all,
             wq_ssem, wq_rsem, wo_ssem, wo_rsem):
        my_pos = lax.axis_index("i")
        left = lax.rem(my_pos + N_DEV - 1, N_DEV)
        right = lax.rem(my_pos + 1, N_DEV)
        jm1 = left
        jp1 = right
        jm2 = lax.rem(my_pos + 2, N_DEV)

        wq_all[pl.ds(my_pos, 1)] = wq_ref[...].astype(BF16)[None]
        wo_all[pl.ds(my_pos, 1)] = wo_ref[...].astype(BF16)[None]

        barrier_sem = pltpu.get_barrier_semaphore()
        for nbr in (left, right):
            pl.semaphore_signal(
                barrier_sem, inc=1,
                device_id=(nbr,), device_id_type=pl.DeviceIdType.MESH,
            )
        pl.semaphore_wait(barrier_sem, 2)

        def copy(buf, slot_idx, ssem, rsem, slot, dev):
            return pltpu.make_async_remote_copy(
                src_ref=buf.at[slot_idx],
                dst_ref=buf.at[slot_idx],
                send_sem=ssem.at[slot],
                recv_sem=rsem.at[slot],
                device_id=(dev,),
                device_id_type=pl.DeviceIdType.MESH,
            )

        q_r0 = copy(wq_all, my_pos, wq_ssem, wq_rsem, 0, right)
        o_l0 = copy(wo_all, my_pos, wo_ssem, wo_rsem, 1, left)
        o_r0 = copy(wo_all, my_pos, wo_ssem, wo_rsem, 0, right)
        q_l0 = copy(wq_all, my_pos, wq_ssem, wq_rsem, 1, left)
        q_r0.start()
        o_l0.start()
        o_r0.start()
        q_l0.start()

        q_recv0 = copy(wq_all, jm1, wq_ssem, wq_rsem, 0, left)
        q_recv1 = copy(wq_all, jp1, wq_ssem, wq_rsem, 1, right)
        q_recv2 = copy(wq_all, jm2, wq_ssem, wq_rsem, 2, left)
        o_recv0 = copy(wo_all, jm1, wo_ssem, wo_rsem, 0, left)
        o_recv1 = copy(wo_all, jp1, wo_ssem, wo_rsem, 1, right)
        o_recv2 = copy(wo_all, jm2, wo_ssem, wo_rsem, 2, right)

        xs = [(x_ref[b] * 0.125).astype(BF16) for b in range(B_LOC)]

        def ctx_phase(j):
            wq_j = wq_all[pl.ds(j, 1)].reshape(D_MODEL, HD_LOC)
            res = []
            for b in range(B_LOC):
                q_blk = lax.dot_general(
                    xs[b], wq_j, (((1,), (0,)), ((), ())),
                    preferred_element_type=F32,
                ).astype(BF16)
                kbj = k_ref[b, :, pl.ds(j * HD_LOC, HD_LOC)]
                vbj = v_ref[b, :, pl.ds(j * HD_LOC, HD_LOC)]
                ctx_t, ctx_b = [], []
                for r in range(H_LOC):
                    sl = slice(r * DH, (r + 1) * DH)
                    k = kbj[:, sl]
                    v = vbj[:, sl]
                    qt = q_blk[:HB, sl]
                    qb = q_blk[HB:, sl]
                    st = lax.dot_general(
                        qt, k[:HB], (((1,), (1,)), ((), ())),
                        preferred_element_type=F32)
                    sb = lax.dot_general(
                        qb, k, (((1,), (1,)), ((), ())),
                        preferred_element_type=F32)
                    et = jnp.exp(st)
                    eb = jnp.exp(sb)
                    rt = 1.0 / jnp.sum(et, axis=-1, keepdims=True)
                    rb = 1.0 / jnp.sum(eb, axis=-1, keepdims=True)
                    ct = lax.dot_general(
                        et.astype(BF16), v[:HB], (((1,), (0,)), ((), ())),
                        preferred_element_type=F32)
                    cb = lax.dot_general(
                        eb.astype(BF16), v, (((1,), (0,)), ((), ())),
                        preferred_element_type=F32)
                    ctx_t.append((ct * rt).astype(BF16))
                    ctx_b.append((cb * rb).astype(BF16))
                res.append((jnp.concatenate(ctx_t, axis=1),
                            jnp.concatenate(ctx_b, axis=1)))
            return res

        def proj(j, ctxs, accs):
            wo_j = wo_all[pl.ds(j, 1)].reshape(HD_LOC, D_MODEL)
            out = []
            for b in range(B_LOC):
                (cat_t, cat_b), (at, ab) = ctxs[b], accs[b]
                out.append((
                    at + lax.dot_general(
                        cat_t, wo_j, (((1,), (0,)), ((), ())),
                        preferred_element_type=F32),
                    ab + lax.dot_general(
                        cat_b, wo_j, (((1,), (0,)), ((), ())),
                        preferred_element_type=F32),
                ))
            return out

        accs = [(jnp.zeros((HB, D_MODEL), F32),
                 jnp.zeros((HB, D_MODEL), F32)) for _ in range(B_LOC)]

        accs = proj(my_pos, ctx_phase(my_pos), accs)

        q_recv0.wait_recv()
        q_f = copy(wq_all, jm1, wq_ssem, wq_rsem, 2, right)
        q_f.start()
        o_recv1.wait_recv()
        o_f = copy(wo_all, jp1, wo_ssem, wo_rsem, 2, left)
        o_f.start()

        c_jm1 = ctx_phase(jm1)
        o_recv0.wait_recv()
        accs = proj(jm1, c_jm1, accs)

        q_recv1.wait_recv()
        accs = proj(jp1, ctx_phase(jp1), accs)

        q_recv2.wait_recv()
        c_jm2 = ctx_phase(jm2)
        o_recv2.wait_recv()
        accs = proj(jm2, c_jm2, accs)

        for b in range(B_LOC):
            out_ref[b] = jnp.concatenate([accs[b][0], accs[b][1]], axis=0)

        for d in (q_r0, q_l0, o_r0, o_l0, q_f, o_f):
            d.wait_send()

    return pl.pallas_call(
        body,
        out_shape=jax.ShapeDtypeStruct((B_LOC, SQ, D_MODEL), F32),
        in_specs=[
            pl.BlockSpec(memory_space=pltpu.VMEM),
            pl.BlockSpec(memory_space=pltpu.VMEM),
            pl.BlockSpec(memory_space=pltpu.VMEM),
            pl.BlockSpec(memory_space=pltpu.VMEM),
            pl.BlockSpec(memory_space=pltpu.VMEM),
        ],
        out_specs=pl.BlockSpec(memory_space=pltpu.VMEM),
        scratch_shapes=[
            pltpu.VMEM((N_DEV, D_MODEL, HD_LOC), BF16),
            pltpu.VMEM((N_DEV, HD_LOC, D_MODEL), BF16),
            pltpu.SemaphoreType.DMA((3,)),
            pltpu.SemaphoreType.DMA((3,)),
            pltpu.SemaphoreType.DMA((3,)),
            pltpu.SemaphoreType.DMA((3,)),
        ],
        compiler_params=pltpu.CompilerParams(collective_id=0),
    )(x, Wq, k_loc, v_loc, Wo)


# device time: 18534 ns/iter; 1.1642x vs baseline; 1.0006x over previous
import jax
import jax.numpy as jnp
from jax import lax
from jax.experimental import pallas as pl
from jax.experimental.pallas import tpu as pltpu

N_DEV = 4
B_LOC = 2
SQ = 128
SKV = 128
HB = 64
HQ = 16
H_LOC = 4
DH = 64
D_MODEL = 512
HD_LOC = H_LOC * DH

BF16 = jnp.bfloat16
F32 = jnp.float32


def kernel(x, Wq, K_ext, V_ext, Wo):
    my = lax.axis_index("i")

    k_loc = lax.dynamic_slice_in_dim(
        K_ext.reshape(8, SKV, HQ * DH), B_LOC * my, B_LOC, axis=0).astype(BF16)
    v_loc = lax.dynamic_slice_in_dim(
        V_ext.reshape(8, SKV, HQ * DH), B_LOC * my, B_LOC, axis=0).astype(BF16)

    def body(x_ref, wq_ref, k_ref, v_ref, wo_ref, out_ref,
             wq_all, wo_all,
             wq_ssem, wq_rsem, wo_ssem, wo_rsem):
        my_pos = lax.axis_index("i")
        left = lax.rem(my_pos + N_DEV - 1, N_DEV)
        right = lax.rem(my_pos + 1, N_DEV)
        jm1 = left
        jp1 = right
        jm2 = lax.rem(my_pos + 2, N_DEV)

        wq_all[pl.ds(my_pos, 1)] = wq_ref[...].astype(BF16)[None]
        wo_all[pl.ds(my_pos, 1)] = wo_ref[...].astype(BF16)[None]

        barrier_sem = pltpu.get_barrier_semaphore()
        for nbr in (left, right):
            pl.semaphore_signal(
                barrier_sem, inc=1,
                device_id=(nbr,), device_id_type=pl.DeviceIdType.MESH,
            )
        pl.semaphore_wait(barrier_sem, 2)

        def copy(buf, slot_idx, ssem, rsem, slot, dev):
            return pltpu.make_async_remote_copy(
                src_ref=buf.at[slot_idx],
                dst_ref=buf.at[slot_idx],
                send_sem=ssem.at[slot],
                recv_sem=rsem.at[slot],
                device_id=(dev,),
                device_id_type=pl.DeviceIdType.MESH,
            )

        q_r0 = copy(wq_all, my_pos, wq_ssem, wq_rsem, 0, right)
        o_l0 = copy(wo_all, my_pos, wo_ssem, wo_rsem, 1, left)
        o_r0 = copy(wo_all, my_pos, wo_ssem, wo_rsem, 0, right)
        q_l0 = copy(wq_all, my_pos, wq_ssem, wq_rsem, 1, left)
        q_r0.start()
        o_l0.start()
        o_r0.start()
        q_l0.start()

        q_recv0 = copy(wq_all, jm1, wq_ssem, wq_rsem, 0, left)
        q_recv1 = copy(wq_all, jp1, wq_ssem, wq_rsem, 1, right)
        q_recv2 = copy(wq_all, jm2, wq_ssem, wq_rsem, 2, left)
        o_recv0 = copy(wo_all, jm1, wo_ssem, wo_rsem, 0, left)
        o_recv1 = copy(wo_all, jp1, wo_ssem, wo_rsem, 1, right)
        o_recv2 = copy(wo_all, jm2, wo_ssem, wo_rsem, 2, right)

        xs = [(x_ref[b] * 0.125).astype(BF16) for b in range(B_LOC)]

        def ctx_phase(j):
            wq_j = wq_all[pl.ds(j, 1)].reshape(D_MODEL, HD_LOC)
            res = []
            for b in range(B_LOC):
                q_blk = lax.dot_general(
                    xs[b], wq_j, (((1,), (0,)), ((), ())),
                    preferred_element_type=F32,
                ).astype(BF16)
                kbj = k_ref[b, :, pl.ds(j * HD_LOC, HD_LOC)]
                vbj = v_ref[b, :, pl.ds(j * HD_LOC, HD_LOC)]
                ctx_t, ctx_b = [], []
                for r in range(H_LOC):
                    sl = slice(r * DH, (r + 1) * DH)
                    k = kbj[:, sl]
                    v = vbj[:, sl]
                    qt = q_blk[:HB, sl]
                    qb = q_blk[HB:, sl]
                    st = lax.dot_general(
                        qt, k[:HB], (((1,), (1,)), ((), ())),
                        preferred_element_type=F32)
                    sb = lax.dot_general(
                        qb, k, (((1,), (1,)), ((), ())),
                        preferred_element_type=F32)
                    et = jnp.exp(st)
                    eb = jnp.exp(sb)
                    rt = 1.0 / jnp.sum(et, axis=-1, keepdims=True)
                    rb = 1.0 / jnp.sum(eb, axis=-1, keepdims=True)
                    ct = lax.dot_general(
                        et.astype(BF16), v[:HB], (((1,), (0,)), ((), ())),
                        preferred_element_type=F32)
                    cb = lax.dot_general(
                        eb.astype(BF16), v, (((1,), (0,)), ((), ())),
                        preferred_element_type=F32)
                    ctx_t.append((ct * rt).astype(BF16))
                    ctx_b.append((cb * rb).astype(BF16))
                res.append((jnp.concatenate(ctx_t, axis=1),
                            jnp.concatenate(ctx_b, axis=1)))
            return res

        def proj(j, ctxs, accs):
            wo_j = wo_all[pl.ds(j, 1)].reshape(HD_LOC, D_MODEL)
            out = []
            for b in range(B_LOC):
                (cat_t, cat_b), (at, ab) = ctxs[b], accs[b]
                out.append((
                    at + lax.dot_general(
                        cat_t, wo_j, (((1,), (0,)), ((), ())),
                        preferred_element_type=F32),
                    ab + lax.dot_general(
                        cat_b, wo_j, (((1,), (0,)), ((), ())),
                        preferred_element_type=F32),
                ))
            return out

        accs = [(jnp.zeros((HB, D_MODEL), F32),
                 jnp.zeros((HB, D_MODEL), F32)) for _ in range(B_LOC)]

        accs = proj(my_pos, ctx_phase(my_pos), accs)

        q_recv0.wait_recv()
        q_f = copy(wq_all, jm1, wq_ssem, wq_rsem, 2, right)
        q_f.start()
        o_recv1.wait_recv()
        o_f = copy(wo_all, jp1, wo_ssem, wo_rsem, 2, left)
        o_f.start()

        c_jm1 = ctx_phase(jm1)
        o_recv0.wait_recv()
        accs = proj(jm1, c_jm1, accs)

        q_recv1.wait_recv()
        accs = proj(jp1, ctx_phase(jp1), accs)

        q_recv2.wait_recv()
        c_jm2 = ctx_phase(jm2)
        o_recv2.wait_recv()
        accs = proj(jm2, c_jm2, accs)

        for b in range(B_LOC):
            out_ref[b] = jnp.concatenate([accs[b][0], accs[b][1]], axis=0)

        for d in (q_r0, q_l0, o_r0, o_l0, q_f, o_f):
            d.wait_send()

    return pl.pallas_call(
        body,
        out_shape=jax.ShapeDtypeStruct((B_LOC, SQ, D_MODEL), F32),
        in_specs=[
            pl.BlockSpec(memory_space=pltpu.VMEM),
            pl.BlockSpec(memory_space=pltpu.VMEM),
            pl.BlockSpec(memory_space=pltpu.VMEM),
            pl.BlockSpec(memory_space=pltpu.VMEM),
            pl.BlockSpec(memory_space=pltpu.VMEM),
        ],
        out_specs=pl.BlockSpec(memory_space=pltpu.VMEM),
        scratch_shapes=[
            pltpu.VMEM((N_DEV, D_MODEL, HD_LOC), BF16),
            pltpu.VMEM((N_DEV, HD_LOC, D_MODEL), BF16),
            pltpu.SemaphoreType.DMA((3,)),
            pltpu.SemaphoreType.DMA((3,)),
            pltpu.SemaphoreType.DMA((3,)),
            pltpu.SemaphoreType.DMA((3,)),
        ],
        compiler_params=pltpu.CompilerParams(collective_id=11),
    )(x, Wq, k_loc, v_loc, Wo)
